# baseline (speedup 1.0000x reference)
import functools

import jax
import jax.numpy as jnp
from jax import lax
from jax.experimental import pallas as pl
from jax.experimental.pallas import tpu as pltpu
from jax.experimental.pallas import tpu_sc as plsc

B = 4096
T = 200
D = 32
L_OUT = 25
NF = 8
NC = 2
NS = 16
NW = NC * NS
B_PER_W = B // NW
INV16 = 1.0 / 16.0
G0, G1 = 104, 96


def _sc_body(xs, xp, st, pt, out, idxs, idxp, rows_s, rows_p, cbuf,
             sem_g, sem_o):
    wid = lax.axis_index("s") * NC + lax.axis_index("c")
    b0 = wid * B_PER_W
    iota = lax.iota(jnp.int32, 16)
    zero = jnp.zeros((16,), jnp.float32)

    pltpu.sync_copy(xs.at[pl.ds(b0 * T, B_PER_W * T)], idxs)
    pltpu.sync_copy(xp.at[pl.ds(b0 * T, B_PER_W * T)], idxp)

    def gather_copies(i, sl):
        return [
            pltpu.make_async_copy(st.at[idxs.at[pl.ds(i * T, G0)]],
                                  rows_s.at[sl, pl.ds(0, G0)], sem_g),
            pltpu.make_async_copy(st.at[idxs.at[pl.ds(i * T + G0, G1)]],
                                  rows_s.at[sl, pl.ds(G0, G1)], sem_g),
            pltpu.make_async_copy(pt.at[idxp.at[pl.ds(i * T, G0)]],
                                  rows_p.at[sl, pl.ds(0, G0)], sem_g),
            pltpu.make_async_copy(pt.at[idxp.at[pl.ds(i * T + G0, G1)]],
                                  rows_p.at[sl, pl.ds(G0, G1)], sem_g),
        ]

    def out_copies(i, sl):
        return [pltpu.make_async_copy(cbuf.at[sl], out.at[b0 + i], sem_o)]

    for cp in gather_copies(0, 0):
        cp.start()

    def per_i(i, carry):
        sl = i & 1
        for cp in gather_copies(i, sl):
            cp.wait()

        @pl.when(i < B_PER_W - 1)
        def _():
            for cp in gather_copies(i + 1, 1 - sl):
                cp.start()

        @pl.when(i >= 2)
        def _():
            for cp in out_copies(i - 2, sl):
                cp.wait()

        def per_k(k, pc):
            p0, p1 = pc
            t0 = 8 * k
            s0 = zero
            s1 = zero
            for j in range(8):
                s0 = s0 + rows_s[sl, t0 + j, 0:16] + rows_p[sl, t0 + j, 0:16]
                s1 = s1 + rows_s[sl, t0 + j, 16:32] + rows_p[sl, t0 + j, 16:32]

            @pl.when(k > 0)
            def _():
                lv = jnp.broadcast_to(k - 1, (16,))
                plsc.store_scatter(cbuf.at[sl], [iota, lv],
                                   (p0 + s0) * INV16)
                plsc.store_scatter(cbuf.at[sl], [iota + 16, lv],
                                   (p1 + s1) * INV16)

            return (s0, s1)

        p0, p1 = lax.fori_loop(0, 25, per_k, (zero, zero))
        e0 = (rows_s[sl, 199, 0:16] + rows_p[sl, 199, 0:16]) * 8.0
        e1 = (rows_s[sl, 199, 16:32] + rows_p[sl, 199, 16:32]) * 8.0
        l24 = jnp.broadcast_to(jnp.int32(24), (16,))
        plsc.store_scatter(cbuf.at[sl], [iota, l24], (p0 + e0) * INV16)
        plsc.store_scatter(cbuf.at[sl], [iota + 16, l24], (p1 + e1) * INV16)

        for cp in out_copies(i, sl):
            cp.start()
        return carry

    lax.fori_loop(0, B_PER_W, per_i, 0)

    for cp in out_copies(B_PER_W - 2, (B_PER_W - 2) & 1):
        cp.wait()
    for cp in out_copies(B_PER_W - 1, (B_PER_W - 1) & 1):
        cp.wait()


_sc_kernel = functools.partial(
    pl.kernel,
    mesh=plsc.VectorSubcoreMesh(core_axis_name="c", subcore_axis_name="s"),
    compiler_params=pltpu.CompilerParams(
        needs_layout_passes=False, use_tc_tiling_on_sc=False),
    out_type=jax.ShapeDtypeStruct((B, D, L_OUT), jnp.float32),
    scratch_types=[
        pltpu.VMEM((B_PER_W * T,), jnp.int32),
        pltpu.VMEM((B_PER_W * T,), jnp.int32),
        pltpu.VMEM((2, T, D), jnp.float32),
        pltpu.VMEM((2, T, D), jnp.float32),
        pltpu.VMEM((2, D, L_OUT), jnp.float32),
        pltpu.SemaphoreType.DMA,
        pltpu.SemaphoreType.DMA,
    ],
)(_sc_body)


GB = 32


def _bc_body(c_ref, o_ref):
    rep = jnp.broadcast_to(c_ref[...][:, None, :, :], (GB, NF, D, L_OUT))
    o_ref[...] = rep.reshape(GB * NF, D, L_OUT)


_bc_kernel = pl.pallas_call(
    _bc_body,
    grid=(B // GB,),
    in_specs=[pl.BlockSpec((GB, D, L_OUT), lambda i: (i, 0, 0))],
    out_specs=pl.BlockSpec((GB * NF, D, L_OUT), lambda i: (i, 0, 0)),
    out_shape=jax.ShapeDtypeStruct((B * NF, D, L_OUT), jnp.float32),
)


@jax.jit
def kernel(x_state, x_pos, state_table, pos_table):
    pooled = _sc_kernel(x_state.reshape(B * T), x_pos.reshape(B * T),
                        state_table, pos_table)
    return _bc_kernel(pooled)


# speedup vs baseline: 1.2267x; 1.2267x over previous
import functools

import jax
import jax.numpy as jnp
from jax import lax
from jax.experimental import pallas as pl
from jax.experimental.pallas import tpu as pltpu
from jax.experimental.pallas import tpu_sc as plsc

B = 4096
T = 200
D = 32
L_OUT = 25
NF = 8
NC = 2
NS = 16
NW = NC * NS
B_PER_W = B // NW
INV16 = 1.0 / 16.0
G0, G1 = 104, 96


def _sc_body(xs, xp, st, pt, out, idxs, idxp, rows_s, rows_p, cbuf,
             sem_g, sem_o):
    wid = lax.axis_index("s") * NC + lax.axis_index("c")
    b0 = wid * B_PER_W
    iota = lax.iota(jnp.int32, 16)
    zero = jnp.zeros((16,), jnp.float32)

    pltpu.sync_copy(xs.at[pl.ds(b0 * T, B_PER_W * T)], idxs)
    pltpu.sync_copy(xp.at[pl.ds(b0 * T, B_PER_W * T)], idxp)

    def gather_copies(i, sl):
        return [
            pltpu.make_async_copy(st.at[idxs.at[pl.ds(i * T, G0)]],
                                  rows_s.at[sl, pl.ds(0, G0)], sem_g),
            pltpu.make_async_copy(st.at[idxs.at[pl.ds(i * T + G0, G1)]],
                                  rows_s.at[sl, pl.ds(G0, G1)], sem_g),
            pltpu.make_async_copy(pt.at[idxp.at[pl.ds(i * T, G0)]],
                                  rows_p.at[sl, pl.ds(0, G0)], sem_g),
            pltpu.make_async_copy(pt.at[idxp.at[pl.ds(i * T + G0, G1)]],
                                  rows_p.at[sl, pl.ds(G0, G1)], sem_g),
        ]

    def out_copies(i, sl):
        return [pltpu.make_async_copy(cbuf.at[sl], out.at[b0 + i], sem_o)]

    for cp in gather_copies(0, 0):
        cp.start()

    def per_i(i, carry):
        sl = i & 1
        for cp in gather_copies(i, sl):
            cp.wait()

        @pl.when(i < B_PER_W - 1)
        def _():
            for cp in gather_copies(i + 1, 1 - sl):
                cp.start()

        @pl.when(i >= 2)
        def _():
            for cp in out_copies(i - 2, sl):
                cp.wait()

        def per_k(k, pc):
            p0, p1 = pc
            t0 = 8 * k
            s0 = zero
            s1 = zero
            for j in range(8):
                s0 = s0 + rows_s[sl, t0 + j, 0:16] + rows_p[sl, t0 + j, 0:16]
                s1 = s1 + rows_s[sl, t0 + j, 16:32] + rows_p[sl, t0 + j, 16:32]

            @pl.when(k > 0)
            def _():
                lv = jnp.broadcast_to(k - 1, (16,))
                plsc.store_scatter(cbuf.at[sl], [iota, lv],
                                   (p0 + s0) * INV16)
                plsc.store_scatter(cbuf.at[sl], [iota + 16, lv],
                                   (p1 + s1) * INV16)

            return (s0, s1)

        p0, p1 = lax.fori_loop(0, 25, per_k, (zero, zero))
        e0 = (rows_s[sl, 199, 0:16] + rows_p[sl, 199, 0:16]) * 8.0
        e1 = (rows_s[sl, 199, 16:32] + rows_p[sl, 199, 16:32]) * 8.0
        l24 = jnp.broadcast_to(jnp.int32(24), (16,))
        plsc.store_scatter(cbuf.at[sl], [iota, l24], (p0 + e0) * INV16)
        plsc.store_scatter(cbuf.at[sl], [iota + 16, l24], (p1 + e1) * INV16)

        for cp in out_copies(i, sl):
            cp.start()
        return carry

    lax.fori_loop(0, B_PER_W, per_i, 0)

    for cp in out_copies(B_PER_W - 2, (B_PER_W - 2) & 1):
        cp.wait()
    for cp in out_copies(B_PER_W - 1, (B_PER_W - 1) & 1):
        cp.wait()


_sc_kernel = functools.partial(
    pl.kernel,
    mesh=plsc.VectorSubcoreMesh(core_axis_name="c", subcore_axis_name="s"),
    compiler_params=pltpu.CompilerParams(
        needs_layout_passes=False, use_tc_tiling_on_sc=False),
    out_type=jax.ShapeDtypeStruct((B, D, L_OUT), jnp.float32),
    scratch_types=[
        pltpu.VMEM((B_PER_W * T,), jnp.int32),
        pltpu.VMEM((B_PER_W * T,), jnp.int32),
        pltpu.VMEM((2, T, D), jnp.float32),
        pltpu.VMEM((2, T, D), jnp.float32),
        pltpu.VMEM((2, D, L_OUT), jnp.float32),
        pltpu.SemaphoreType.DMA,
        pltpu.SemaphoreType.DMA,
    ],
)(_sc_body)


@jax.jit
def kernel(x_state, x_pos, state_table, pos_table):
    pooled = _sc_kernel(x_state.reshape(B * T), x_pos.reshape(B * T),
                        state_table, pos_table)
    rep = jnp.broadcast_to(pooled[:, None, :, :], (B, NF, D, L_OUT))
    return rep.reshape(B * NF, D, L_OUT)


# speedup vs baseline: 1.3156x; 1.0725x over previous
import functools

import jax
import jax.numpy as jnp
from jax import lax
from jax.experimental import pallas as pl
from jax.experimental.pallas import tpu as pltpu
from jax.experimental.pallas import tpu_sc as plsc

B = 4096
T = 200
D = 32
L_OUT = 25
NF = 8
NC = 2
NS = 16
NW = NC * NS
B_PER_W = B // NW
INV16 = 1.0 / 16.0
G0, G1 = 104, 96


def _sc_body(xs, xp, st, pt, out, idxs, idxp, rows_s, rows_p, cbuf,
             sem_g, sem_o):
    wid = lax.axis_index("s") * NC + lax.axis_index("c")
    b0 = wid * B_PER_W
    iota = lax.iota(jnp.int32, 16)
    d25_lo = iota * L_OUT
    d25_hi = (iota + 16) * L_OUT
    zero = jnp.zeros((16,), jnp.float32)

    pltpu.sync_copy(xs.at[pl.ds(b0 * T, B_PER_W * T)], idxs)
    pltpu.sync_copy(xp.at[pl.ds(b0 * T, B_PER_W * T)], idxp)

    def gather_copies(i, sl):
        return [
            pltpu.make_async_copy(st.at[idxs.at[pl.ds(i * T, G0)]],
                                  rows_s.at[sl, pl.ds(0, G0)], sem_g),
            pltpu.make_async_copy(st.at[idxs.at[pl.ds(i * T + G0, G1)]],
                                  rows_s.at[sl, pl.ds(G0, G1)], sem_g),
            pltpu.make_async_copy(pt.at[idxp.at[pl.ds(i * T, G0)]],
                                  rows_p.at[sl, pl.ds(0, G0)], sem_g),
            pltpu.make_async_copy(pt.at[idxp.at[pl.ds(i * T + G0, G1)]],
                                  rows_p.at[sl, pl.ds(G0, G1)], sem_g),
        ]

    def out_copies(i, sl):
        return [pltpu.make_async_copy(cbuf.at[sl], out.at[b0 + i], sem_o)]

    for cp in gather_copies(0, 0):
        cp.start()

    def per_i(i, carry):
        sl = i & 1
        for cp in gather_copies(i, sl):
            cp.wait()

        @pl.when(i < B_PER_W - 1)
        def _():
            for cp in gather_copies(i + 1, 1 - sl):
                cp.start()

        @pl.when(i >= 2)
        def _():
            for cp in out_copies(i - 2, sl):
                cp.wait()

        def per_k(k, pc):
            p0, p1 = pc
            t0 = 8 * k
            s0 = zero
            s1 = zero
            for j in range(8):
                s0 = s0 + rows_s[sl, t0 + j, 0:16] + rows_p[sl, t0 + j, 0:16]
                s1 = s1 + rows_s[sl, t0 + j, 16:32] + rows_p[sl, t0 + j, 16:32]

            @pl.when(k > 0)
            def _():
                lv = jnp.broadcast_to(k - 1, (16,))
                plsc.store_scatter(cbuf.at[sl], [d25_lo + lv],
                                   (p0 + s0) * INV16)
                plsc.store_scatter(cbuf.at[sl], [d25_hi + lv],
                                   (p1 + s1) * INV16)

            return (s0, s1)

        p0, p1 = lax.fori_loop(0, 25, per_k, (zero, zero))
        e0 = (rows_s[sl, 199, 0:16] + rows_p[sl, 199, 0:16]) * 8.0
        e1 = (rows_s[sl, 199, 16:32] + rows_p[sl, 199, 16:32]) * 8.0
        plsc.store_scatter(cbuf.at[sl], [d25_lo + 24], (p0 + e0) * INV16)
        plsc.store_scatter(cbuf.at[sl], [d25_hi + 24], (p1 + e1) * INV16)

        for cp in out_copies(i, sl):
            cp.start()
        return carry

    lax.fori_loop(0, B_PER_W, per_i, 0)

    for cp in out_copies(B_PER_W - 2, (B_PER_W - 2) & 1):
        cp.wait()
    for cp in out_copies(B_PER_W - 1, (B_PER_W - 1) & 1):
        cp.wait()


_sc_kernel = functools.partial(
    pl.kernel,
    mesh=plsc.VectorSubcoreMesh(core_axis_name="c", subcore_axis_name="s"),
    compiler_params=pltpu.CompilerParams(
        needs_layout_passes=False, use_tc_tiling_on_sc=False),
    out_type=jax.ShapeDtypeStruct((B, D * L_OUT), jnp.float32),
    scratch_types=[
        pltpu.VMEM((B_PER_W * T,), jnp.int32),
        pltpu.VMEM((B_PER_W * T,), jnp.int32),
        pltpu.VMEM((2, T, D), jnp.float32),
        pltpu.VMEM((2, T, D), jnp.float32),
        pltpu.VMEM((2, D * L_OUT), jnp.float32),
        pltpu.SemaphoreType.DMA,
        pltpu.SemaphoreType.DMA,
    ],
)(_sc_body)


@jax.jit
def kernel(x_state, x_pos, state_table, pos_table):
    pooled = _sc_kernel(x_state.reshape(B * T), x_pos.reshape(B * T),
                        state_table, pos_table)
    c = pooled.reshape(B, D, L_OUT)
    rep = jnp.broadcast_to(c[:, None, :, :], (B, NF, D, L_OUT))
    return rep.reshape(B * NF, D, L_OUT)
